# baseline (speedup 1.0000x reference)
import functools

import jax
import jax.numpy as jnp
from jax import lax
from jax.experimental import pallas as pl
from jax.experimental.pallas import tpu as pltpu
from jax.experimental.pallas import tpu_sc as plsc

_NC = 2
_NS = 16
_NW = _NC * _NS

_CHUNK = 128


def _emb_body(n_chunks, table_hbm, idx_hbm, out_hbm, idx_v, rows_v, gsem):
    wid = lax.axis_index("s") * _NC + lax.axis_index("c")
    pltpu.sync_copy(idx_hbm.at[wid], idx_v)

    def chunk(g, _):
        off = pl.multiple_of(g * _CHUNK, _CHUNK)
        pltpu.async_copy(table_hbm.at[idx_v.at[g]], rows_v, gsem).wait()
        pltpu.sync_copy(rows_v, out_hbm.at[wid, pl.ds(off, _CHUNK)])
        return 0

    lax.fori_loop(0, n_chunks, chunk, 0)


@functools.partial(jax.jit, static_argnums=(2, 3))
def _emb_lookup(weights, flat_idx, n_chunks, dim):
    total = flat_idx.shape[0]
    per_w = total // _NW
    idx3d = flat_idx.reshape(_NW, n_chunks, _CHUNK)
    mesh = plsc.VectorSubcoreMesh(core_axis_name="c", subcore_axis_name="s")
    run = pl.kernel(
        functools.partial(_emb_body, n_chunks),
        out_type=jax.ShapeDtypeStruct((_NW, per_w, dim), jnp.float32),
        mesh=mesh,
        scratch_types=[
            pltpu.VMEM((n_chunks, _CHUNK), jnp.int32),
            pltpu.VMEM((_CHUNK, dim), jnp.float32),
            pltpu.SemaphoreType.DMA,
        ],
        compiler_params=pltpu.CompilerParams(use_tc_tiling_on_sc=False),
    )
    out = run(weights, idx3d)
    return out.reshape(total, dim)


def kernel(weights, token_ids):
    batch, fields = token_ids.shape
    dim = weights.shape[1]
    total = batch * fields
    per_w = total // _NW
    n_chunks = per_w // _CHUNK
    flat = token_ids.reshape(total)
    out = _emb_lookup(weights, flat, n_chunks, dim)
    return out.reshape(batch, fields, dim)


# speedup vs baseline: 1.0748x; 1.0748x over previous
import functools

import jax
import jax.numpy as jnp
from jax import lax
from jax.experimental import pallas as pl
from jax.experimental.pallas import tpu as pltpu
from jax.experimental.pallas import tpu_sc as plsc

_NC = 2
_NS = 16
_NW = _NC * _NS

_CHUNK = 128
_SUPER = 4
_GROUP = _CHUNK * _SUPER


def _emb_body(n_groups, dim, table_hbm, idx_hbm, out_hbm,
              idx_v, buf0, buf1, gs0, gs1, os0, os1):
    wid = lax.axis_index("s") * _NC + lax.axis_index("c")
    pltpu.sync_copy(idx_hbm.at[wid], idx_v)

    bufs = (buf0, buf1)
    gsems = (gs0, gs1)
    osems = (os0, os1)

    def fire(g):
        b = g % 2
        return [
            pltpu.async_copy(
                table_hbm.at[idx_v.at[g * _SUPER + k]],
                bufs[b].at[pl.ds(k * _CHUNK, _CHUNK)],
                gsems[b],
            )
            for k in range(_SUPER)
        ]

    pend = {0: fire(0), 1: fire(1)}
    tail = []
    for g in range(n_groups):
        b = g % 2
        for cp in pend[b]:
            cp.wait()
        ocp = pltpu.async_copy(
            bufs[b], out_hbm.at[wid, pl.ds(g * _GROUP, _GROUP)], osems[b]
        )
        if g + 2 < n_groups:
            ocp.wait()
            pend[b] = fire(g + 2)
        else:
            tail.append(ocp)
    for ocp in tail:
        ocp.wait()


@functools.partial(jax.jit, static_argnums=(2, 3))
def _emb_lookup(weights, flat_idx, n_groups, dim):
    total = flat_idx.shape[0]
    per_w = total // _NW
    idx3d = flat_idx.reshape(_NW, per_w // _CHUNK, _CHUNK)
    mesh = plsc.VectorSubcoreMesh(core_axis_name="c", subcore_axis_name="s")
    run = pl.kernel(
        functools.partial(_emb_body, n_groups, dim),
        out_type=jax.ShapeDtypeStruct((_NW, per_w, dim), jnp.float32),
        mesh=mesh,
        scratch_types=[
            pltpu.VMEM((per_w // _CHUNK, _CHUNK), jnp.int32),
            pltpu.VMEM((_GROUP, dim), jnp.float32),
            pltpu.VMEM((_GROUP, dim), jnp.float32),
            pltpu.SemaphoreType.DMA,
            pltpu.SemaphoreType.DMA,
            pltpu.SemaphoreType.DMA,
            pltpu.SemaphoreType.DMA,
        ],
        compiler_params=pltpu.CompilerParams(use_tc_tiling_on_sc=False),
    )
    out = run(weights, idx3d)
    return out.reshape(total, dim)


def kernel(weights, token_ids):
    batch, fields = token_ids.shape
    dim = weights.shape[1]
    total = batch * fields
    per_w = total // _NW
    n_groups = per_w // _GROUP
    flat = token_ids.reshape(total)
    out = _emb_lookup(weights, flat, n_groups, dim)
    return out.reshape(batch, fields, dim)


# speedup vs baseline: 1.1884x; 1.1057x over previous
import functools

import jax
import jax.numpy as jnp
from jax import lax
from jax.experimental import pallas as pl
from jax.experimental.pallas import tpu as pltpu
from jax.experimental.pallas import tpu_sc as plsc
from jax.experimental.layout import Format, Layout

_NC = 2
_NS = 16
_NW = _NC * _NS

_CHUNK = 128
_SUPER = 4
_GROUP = _CHUNK * _SUPER


def _emb_body(n_groups, dim, table_hbm, idx_hbm, out_hbm,
              idx_v, buf0, buf1, gs0, gs1, os0, os1):
    wid = lax.axis_index("s") * _NC + lax.axis_index("c")
    pltpu.sync_copy(idx_hbm.at[wid], idx_v)

    bufs = (buf0, buf1)
    gsems = (gs0, gs1)
    osems = (os0, os1)

    def fire(g):
        b = g % 2
        return [
            pltpu.async_copy(
                table_hbm.at[idx_v.at[g * _SUPER + k]],
                bufs[b].at[pl.ds(k * _CHUNK, _CHUNK)],
                gsems[b],
            )
            for k in range(_SUPER)
        ]

    pend = {0: fire(0), 1: fire(1)}
    tail = []
    for g in range(n_groups):
        b = g % 2
        for cp in pend[b]:
            cp.wait()
        ocp = pltpu.async_copy(
            bufs[b], out_hbm.at[wid, pl.ds(g * _GROUP, _GROUP)], osems[b]
        )
        if g + 2 < n_groups:
            ocp.wait()
            pend[b] = fire(g + 2)
        else:
            tail.append(ocp)
    for ocp in tail:
        ocp.wait()


@functools.partial(jax.jit, static_argnums=(2, 3))
def _emb_lookup(weights, flat_idx, n_groups, dim):
    total = flat_idx.shape[0]
    per_w = total // _NW
    idx3d = flat_idx.reshape(_NW, per_w // _CHUNK, _CHUNK)
    mesh = plsc.VectorSubcoreMesh(core_axis_name="c", subcore_axis_name="s")
    run = pl.kernel(
        functools.partial(_emb_body, n_groups, dim),
        out_type=jax.ShapeDtypeStruct((_NW, per_w, dim), jnp.float32),
        mesh=mesh,
        scratch_types=[
            pltpu.VMEM((per_w // _CHUNK, _CHUNK), jnp.int32),
            pltpu.VMEM((_GROUP, dim), jnp.float32),
            pltpu.VMEM((_GROUP, dim), jnp.float32),
            pltpu.SemaphoreType.DMA,
            pltpu.SemaphoreType.DMA,
            pltpu.SemaphoreType.DMA,
            pltpu.SemaphoreType.DMA,
        ],
        compiler_params=pltpu.CompilerParams(use_tc_tiling_on_sc=False),
    )
    out = run(weights, idx3d)
    return out.reshape(total, dim)


def kernel(weights, token_ids):
    batch, fields = token_ids.shape
    dim = weights.shape[1]
    total = batch * fields
    per_w = total // _NW
    n_groups = per_w // _GROUP
    flat = token_ids.reshape(total)
    out = _emb_lookup(weights, flat, n_groups, dim)
    out3d = out.reshape(batch, fields, dim)
    return jax.experimental.layout.with_layout_constraint(
        out3d, Layout((0, 1, 2))
    )


# speedup vs baseline: 1.4379x; 1.2099x over previous
import functools

import jax
import jax.numpy as jnp
from jax import lax
from jax.experimental import pallas as pl
from jax.experimental.pallas import tpu as pltpu
from jax.experimental.pallas import tpu_sc as plsc
from jax.experimental.layout import Layout

_NC = 2
_NS = 16
_NW = _NC * _NS

_CHUNK = 104
_SUPER = 8
_GROUP = _CHUNK * _SUPER


def _emb_body(n_groups, fields, dim, table_hbm, idx_hbm, out_hbm,
              idx_v, buf0, buf1, gs0, gs1, os0, os1):
    wid = lax.axis_index("s") * _NC + lax.axis_index("c")
    per_w = n_groups * _GROUP
    b_per_g = _GROUP // fields
    base_k = wid * per_w
    base_b = wid * (per_w // fields)
    pltpu.sync_copy(idx_hbm.at[pl.ds(base_k, per_w)], idx_v)

    bufs = (buf0, buf1)
    gsems = (gs0, gs1)
    osems = (os0, os1)

    def fire(g):
        b = g % 2
        return [
            pltpu.async_copy(
                table_hbm.at[idx_v.at[pl.ds(g * _GROUP + k * _CHUNK, _CHUNK)]],
                bufs[b].at[pl.ds(k * _CHUNK, _CHUNK)],
                gsems[b],
            )
            for k in range(_SUPER)
        ]

    def drain_out(g):
        b = g % 2
        return [
            pltpu.async_copy(
                bufs[b].at[pl.ds(j * fields, fields)],
                out_hbm.at[base_b + g * b_per_g + j],
                osems[b],
            )
            for j in range(b_per_g)
        ]

    pend = {0: fire(0), 1: fire(1)}
    tail = []
    for g in range(n_groups):
        b = g % 2
        for cp in pend[b]:
            cp.wait()
        ocps = drain_out(g)
        if g + 2 < n_groups:
            for ocp in ocps:
                ocp.wait()
            pend[b] = fire(g + 2)
        else:
            tail.extend(ocps)
    for ocp in tail:
        ocp.wait()


@functools.partial(jax.jit, static_argnums=(2, 3, 4))
def _emb_lookup(weights, flat_idx, n_groups, fields, dim):
    total = flat_idx.shape[0]
    batch = total // fields
    mesh = plsc.VectorSubcoreMesh(core_axis_name="c", subcore_axis_name="s")
    run = pl.kernel(
        functools.partial(_emb_body, n_groups, fields, dim),
        out_type=jax.ShapeDtypeStruct((batch, fields, dim), jnp.float32),
        mesh=mesh,
        scratch_types=[
            pltpu.VMEM((total // _NW,), jnp.int32),
            pltpu.VMEM((_GROUP, dim), jnp.float32),
            pltpu.VMEM((_GROUP, dim), jnp.float32),
            pltpu.SemaphoreType.DMA,
            pltpu.SemaphoreType.DMA,
            pltpu.SemaphoreType.DMA,
            pltpu.SemaphoreType.DMA,
        ],
        compiler_params=pltpu.CompilerParams(use_tc_tiling_on_sc=False),
    )
    wrow = jax.experimental.layout.with_layout_constraint(
        weights, Layout((0, 1), tiling=((8, 64),))
    )
    return run(wrow, flat_idx)


def kernel(weights, token_ids):
    batch, fields = token_ids.shape
    dim = weights.shape[1]
    total = batch * fields
    per_w = total // _NW
    n_groups = per_w // _GROUP
    flat = token_ids.reshape(total)
    return _emb_lookup(weights, flat, n_groups, fields, dim)
